# baseline (speedup 1.0000x reference)
import jax
import jax.numpy as jnp
from jax.experimental import pallas as pl


def _router_kernel(x_ref, w_ref, b_ref, logits_ref, probs_ref, zacc_ref):
    x = x_ref[...]
    w = w_ref[...]
    logits = jax.lax.dot_general(
        x, w, (((1,), (1,)), ((), ())),
        preferred_element_type=jnp.float32,
    )
    logits = logits + b_ref[...]
    m = jnp.max(logits, axis=-1, keepdims=True)
    e = jnp.exp(logits - m)
    s = jnp.sum(e, axis=-1, keepdims=True)
    logits_ref[...] = logits
    probs_ref[...] = e / s
    log_z = m + jnp.log(s)
    zacc_ref[...] = jnp.sum(log_z * log_z).reshape(1, 1, 1)


def kernel(token_inputs, W, b, expert_capacity):
    G, T, D = token_inputs.shape
    E = W.shape[0]
    N = G * T
    x = token_inputs.reshape(N, D)

    BT = 1024
    grid = (N // BT,)

    logits, probs, zacc = pl.pallas_call(
        _router_kernel,
        grid=grid,
        in_specs=[
            pl.BlockSpec((BT, D), lambda i: (i, 0)),
            pl.BlockSpec((E, D), lambda i: (0, 0)),
            pl.BlockSpec((1, E), lambda i: (0, 0)),
        ],
        out_specs=[
            pl.BlockSpec((BT, E), lambda i: (i, 0)),
            pl.BlockSpec((BT, E), lambda i: (i, 0)),
            pl.BlockSpec((1, 1, 1), lambda i: (i, 0, 0)),
        ],
        out_shape=[
            jax.ShapeDtypeStruct((N, E), jnp.float32),
            jax.ShapeDtypeStruct((N, E), jnp.float32),
            jax.ShapeDtypeStruct((N // BT, 1, 1), jnp.float32),
        ],
    )(x, W, b.reshape(1, E))

    router_logits = logits.reshape(G, T, E)
    router_probabilities = probs.reshape(G, T, E)
    router_z_loss = jnp.sum(zacc) / (G * T)
    router_causal_loss = jnp.asarray(0.0, dtype=jnp.float32)
    return (router_logits, router_probabilities, router_z_loss, router_causal_loss)


# speedup vs baseline: 1.0026x; 1.0026x over previous
import jax
import jax.numpy as jnp
from jax.experimental import pallas as pl


def _router_kernel(x_ref, w_ref, b_ref, logits_ref, probs_ref, zacc_ref):
    x = x_ref[...].astype(jnp.bfloat16)
    w = w_ref[...].astype(jnp.bfloat16)
    logits = jax.lax.dot_general(
        x, w, (((1,), (1,)), ((), ())),
        preferred_element_type=jnp.float32,
    )
    logits = logits + b_ref[...]
    m = jnp.max(logits, axis=-1, keepdims=True)
    e = jnp.exp(logits - m)
    s = jnp.sum(e, axis=-1, keepdims=True)
    logits_ref[...] = logits
    probs_ref[...] = e / s
    log_z = m + jnp.log(s)
    zacc_ref[...] = jnp.sum(log_z * log_z).reshape(1, 1, 1)


def kernel(token_inputs, W, b, expert_capacity):
    G, T, D = token_inputs.shape
    E = W.shape[0]
    N = G * T
    x = token_inputs.reshape(N, D)

    BT = 1024
    grid = (N // BT,)

    logits, probs, zacc = pl.pallas_call(
        _router_kernel,
        grid=grid,
        in_specs=[
            pl.BlockSpec((BT, D), lambda i: (i, 0)),
            pl.BlockSpec((E, D), lambda i: (0, 0)),
            pl.BlockSpec((1, E), lambda i: (0, 0)),
        ],
        out_specs=[
            pl.BlockSpec((BT, E), lambda i: (i, 0)),
            pl.BlockSpec((BT, E), lambda i: (i, 0)),
            pl.BlockSpec((1, 1, 1), lambda i: (i, 0, 0)),
        ],
        out_shape=[
            jax.ShapeDtypeStruct((N, E), jnp.float32),
            jax.ShapeDtypeStruct((N, E), jnp.float32),
            jax.ShapeDtypeStruct((N // BT, 1, 1), jnp.float32),
        ],
    )(x, W, b.reshape(1, E))

    router_logits = logits.reshape(G, T, E)
    router_probabilities = probs.reshape(G, T, E)
    router_z_loss = jnp.sum(zacc) / (G * T)
    router_causal_loss = jnp.asarray(0.0, dtype=jnp.float32)
    return (router_logits, router_probabilities, router_z_loss, router_causal_loss)


# speedup vs baseline: 1.0290x; 1.0263x over previous
import jax
import jax.numpy as jnp
from jax.experimental import pallas as pl


def _router_kernel(xa_ref, xb_ref, w_ref, b_ref, logits_ref, probs_ref, zacc_ref):
    w = w_ref[...]
    ha = w.shape[1] // 2
    la = jax.lax.dot_general(
        xa_ref[...], w[:, :ha], (((1,), (1,)), ((), ())),
        preferred_element_type=jnp.float32,
    )
    lb = jax.lax.dot_general(
        xb_ref[...], w[:, ha:], (((1,), (1,)), ((), ())),
        preferred_element_type=jnp.float32,
    )
    logits = la + lb + b_ref[...]
    m = jnp.max(logits, axis=-1, keepdims=True)
    e = jnp.exp(logits - m)
    s = jnp.sum(e, axis=-1, keepdims=True)
    logits_ref[...] = logits
    probs_ref[...] = e / s
    log_z = m + jnp.log(s)
    part = jnp.sum(log_z * log_z, keepdims=True)

    @pl.when(pl.program_id(0) == 0)
    def _init():
        zacc_ref[...] = jnp.zeros_like(zacc_ref)

    zacc_ref[...] += part


def kernel(token_inputs, W, b, expert_capacity):
    G, T, D = token_inputs.shape
    E = W.shape[0]
    N = G * T
    x = token_inputs.reshape(N, D)

    BT = 1024
    grid = (N // BT,)

    logits, probs, zacc = pl.pallas_call(
        _router_kernel,
        grid=grid,
        in_specs=[
            pl.BlockSpec((BT, D // 2), lambda i: (i, 0)),
            pl.BlockSpec((BT, D // 2), lambda i: (i, 1)),
            pl.BlockSpec((E, D), lambda i: (0, 0)),
            pl.BlockSpec((1, E), lambda i: (0, 0)),
        ],
        out_specs=[
            pl.BlockSpec((BT, E), lambda i: (i, 0)),
            pl.BlockSpec((BT, E), lambda i: (i, 0)),
            pl.BlockSpec((1, 1), lambda i: (0, 0)),
        ],
        out_shape=[
            jax.ShapeDtypeStruct((N, E), jnp.float32),
            jax.ShapeDtypeStruct((N, E), jnp.float32),
            jax.ShapeDtypeStruct((1, 1), jnp.float32),
        ],
    )(x, x, W, b.reshape(1, E))

    router_logits = logits.reshape(G, T, E)
    router_probabilities = probs.reshape(G, T, E)
    router_z_loss = zacc[0, 0] / (G * T)
    router_causal_loss = jnp.asarray(0.0, dtype=jnp.float32)
    return (router_logits, router_probabilities, router_z_loss, router_causal_loss)
